# baseline (speedup 1.0000x reference)
import jax
import jax.numpy as jnp
from jax.experimental import pallas as pl
from jax.experimental.pallas import tpu as pltpu

_N = 4096
_H = 128
_TM = 512
_MB = _N // _TM
_TF = 256
_C0 = 41
_D0 = 58
_G = _D0 + 24 + 1
_BF16 = jnp.bfloat16
_F32 = jnp.float32


def _normalize(x):
    n = jnp.sqrt(jnp.sum(x * x, axis=1, keepdims=True))
    return x / jnp.maximum(n, 1e-12)


def _split(a):
    hi = a.astype(_BF16)
    lo = (a - hi.astype(_F32)).astype(_BF16)
    return hi, lo


def _dot3(a, wh, wl):
    ah, al = _split(a)
    return (jnp.dot(ah, wh, preferred_element_type=_F32)
            + jnp.dot(al, wh, preferred_element_type=_F32)
            + jnp.dot(ah, wl, preferred_element_type=_F32))


def _score(x, l1h_ref, l1l_ref, l1b_ref, l2h_ref, l2l_ref, l2b_ref,
           l3r_ref, l3b_ref):
    h = _dot3(x, l1h_ref[...], l1l_ref[...]) + l1b_ref[...]
    h = jnp.maximum(h, 0.0)
    h = _dot3(h, l2h_ref[...], l2l_ref[...]) + l2b_ref[...]
    h = jnp.maximum(h, 0.0)
    return jnp.sum(h * l3r_ref[...], axis=1, keepdims=True) + l3b_ref[...]


def _tailmeta(g):
    inA = (g >= 3) & (g <= 17) & (g % 2 == 1)
    inB = (g >= 18) & (g < 42)
    inC = (g >= 43) & (g <= 57) & (g % 2 == 1)
    inD = (g >= _D0 + 1) & (g < _G)
    lp = jnp.where(inB, 1 + jnp.clip(g - 18, 0, 23) // _MB,
                   jnp.where(inD, 1 + jnp.clip(g - _D0 - 1, 0, 23) // _MB, 0))
    mp = jnp.where(inA, jnp.clip((g - 3) // 2, 0, _MB - 1),
                   jnp.where(inB, jnp.clip(g - 18, 0, 23) % _MB,
                             jnp.where(inC, jnp.clip((g - 43) // 2, 0, _MB - 1),
                                       jnp.clip(g - _D0 - 1, 0, 23) % _MB)))
    bp = jnp.where(inC | inD, 1, 0)
    return lp, mp, bp, inA | inB | inC | inD


def _body(adj1_ref, adj2_ref, w1_ref, wn_ref, l1h_ref, l1l_ref, l1b_ref,
          l2h_ref, l2l_ref, l2b_ref, l3r_ref, l3b_ref,
          prod_ref, adj_s, x_s, rhs_s, s_s, cs_s):
    g = pl.program_id(0)

    @pl.when(g < 16)
    def _():
        adj_s[pl.ds(g * _TF, _TF), :] = (adj1_ref[...] - 0.5).astype(_BF16)

    @pl.when(g == 0)
    def _():
        cs_s[16:17, :] = jnp.sum(w1_ref[...], axis=0, keepdims=True)

    lp, mp, bp, tv = _tailmeta(g)
    x = x_s[...]
    xn = _normalize(x)
    xs = jnp.where(lp <= 2, xn, x)
    sc = _score(xs, l1h_ref, l1l_ref, l1b_ref, l2h_ref, l2l_ref, l2b_ref,
                l3r_ref, l3b_ref)
    r = jnp.dot(xn.astype(_BF16), wn_ref[0].astype(_BF16),
                preferred_element_type=_F32)
    kslot = (bp * _MB + mp) * _TM

    @pl.when(tv & (lp == 0))
    def _():
        s_s[pl.ds(kslot, _TM), :] = sc

    @pl.when(tv & (lp >= 1) & (lp <= 2))
    def _():
        s_s[pl.ds(kslot, _TM), :] = s_s[pl.ds(kslot, _TM), :] + sc

    @pl.when(tv & (lp <= 2))
    def _():
        woff = ((lp + 1) % 2) * _N + mp * _TM
        rhs_s[pl.ds(woff, _TM), :] = r.astype(_BF16)
        csp = ((lp + 1) % 2) * 8
        rcs = jnp.sum(r, axis=0, keepdims=True)

        @pl.when(mp == 0)
        def _():
            cs_s[pl.ds(csp, 1), :] = rcs

        @pl.when(mp >= 1)
        def _():
            cs_s[pl.ds(csp, 1), :] = cs_s[pl.ds(csp, 1), :] + rcs

    @pl.when(tv & (lp == 3))
    def _():
        tot = s_s[pl.ds(kslot, _TM), :] + sc

        @pl.when(bp == 0)
        def _():
            s_s[pl.ds(kslot, _TM), :] = tot

        @pl.when(bp == 1)
        def _():
            prod_ref[...] = tot * s_s[pl.ds(mp * _TM, _TM), :]

    @pl.when((g >= 2) & (g <= 16) & (g % 2 == 0))
    def _():
        m = jnp.clip((g - 2) // 2, 0, _MB - 1)
        adjb = adj_s[pl.ds(m * _TM, _TM), :]
        x_s[...] = jnp.maximum(
            jnp.dot(adjb, w1_ref[...].astype(_BF16),
                    preferred_element_type=_F32) + 0.5 * cs_s[16:17, :], 0.0)

    @pl.when((g >= _C0) & (g < _C0 + 16))
    def _():
        j = jnp.clip(g - _C0, 0, 15)
        adjh = (adj2_ref[...] - 0.5).astype(_BF16)
        adj_s[pl.ds(j * _TF, _TF), :] = adjh
        xh = jnp.maximum(
            jnp.dot(adjh, w1_ref[...].astype(_BF16),
                    preferred_element_type=_F32) + 0.5 * cs_s[16:17, :], 0.0)
        x_s[pl.ds((j % 2) * _TF, _TF), :] = xh

    inB = (g >= 17) & (g < _C0)
    inD = (g >= _D0) & (g < _D0 + 24)

    @pl.when(inB | inD)
    def _():
        q = jnp.where(inB, jnp.clip(g - 17, 0, 23), jnp.clip(g - _D0, 0, 23))
        l = 1 + q // _MB
        m = q % _MB
        adjb = adj_s[pl.ds(m * _TM, _TM), :]
        rhs = rhs_s[pl.ds((l % 2) * _N, _N), :]
        x_s[...] = jnp.maximum(
            jnp.dot(adjb, rhs, preferred_element_type=_F32)
            + 0.5 * cs_s[pl.ds((l % 2) * 8, 1), :], 0.0)


def _wn_idx(g):
    inB = (g >= 18) & (g < 42)
    inD = (g >= _D0 + 1)
    lw = jnp.where(inB, 1 + jnp.clip(g - 18, 0, 23) // _MB,
                   jnp.where(inD, 1 + jnp.clip(g - _D0 - 1, 0, 23) // _MB, 0))
    return jnp.clip(lw, 0, 2)


def kernel(adjacent_1, adjacent_2, W1, W2, W3, W4,
           lin1_w, lin1_b, lin2_w, lin2_b, lin3_w, lin3_b):
    a1_spec = pl.BlockSpec((_TF, _N), lambda g: (jnp.clip(g, 0, 15), 0))
    a2_spec = pl.BlockSpec((_TF, _N), lambda g: (jnp.clip(g - _C0, 0, 15), 0))
    wn_spec = pl.BlockSpec((1, _H, _H), lambda g: (_wn_idx(g), 0, 0))
    full = lambda shape: pl.BlockSpec(shape, lambda g: (0, 0))
    prod_spec = pl.BlockSpec(
        (_TM, 1), lambda g: (jnp.clip(g - (_D0 + 17), 0, _MB - 1), 0))
    wstack = jnp.stack([W2, W3, W4])
    l1h = lin1_w.astype(_BF16)
    l1l = (lin1_w - l1h.astype(_F32)).astype(_BF16)
    l2h = lin2_w.astype(_BF16)
    l2l = (lin2_w - l2h.astype(_F32)).astype(_BF16)
    lins = (
        l1h,
        l1l,
        lin1_b.reshape(1, 2 * _H),
        l2h,
        l2l,
        lin2_b.reshape(1, 2 * _H),
        lin3_w.reshape(1, 2 * _H),
        lin3_b.reshape(1, 1),
    )
    return pl.pallas_call(
        _body,
        grid=(_G,),
        in_specs=[a1_spec, a2_spec, full((_N, _H)), wn_spec,
                  full((_H, 2 * _H)), full((_H, 2 * _H)),
                  full((1, 2 * _H)),
                  full((2 * _H, 2 * _H)), full((2 * _H, 2 * _H)),
                  full((1, 2 * _H)),
                  full((1, 2 * _H)), full((1, 1))],
        out_specs=prod_spec,
        out_shape=jax.ShapeDtypeStruct((_N, 1), _F32),
        compiler_params=pltpu.CompilerParams(
            vmem_limit_bytes=100 * 1024 * 1024),
        scratch_shapes=[
            pltpu.VMEM((_N, _N), _BF16),
            pltpu.VMEM((_TM, _H), _F32),
            pltpu.VMEM((2 * _N, _H), _BF16),
            pltpu.VMEM((2 * _N, 1), _F32),
            pltpu.VMEM((24, _H), _F32),
        ],
    )(adjacent_1, adjacent_2, W1, wstack, *lins)


# speedup vs baseline: 1.2274x; 1.2274x over previous
import jax
import jax.numpy as jnp
from jax.experimental import pallas as pl
from jax.experimental.pallas import tpu as pltpu

_N = 4096
_H = 128
_TM = 512
_MB = _N // _TM
_TF = 256
_C0 = 41
_D0 = 58
_G = _D0 + 24 + 1
_BF16 = jnp.bfloat16
_F32 = jnp.float32


def _normalize(x):
    n = jnp.sqrt(jnp.sum(x * x, axis=1, keepdims=True))
    return x / jnp.maximum(n, 1e-12)


def _score(x, l1h_ref, l1b_ref, l2h_ref, l2b_ref, l3r_ref, l3b_ref):
    h = jnp.dot(x.astype(_BF16), l1h_ref[...],
                preferred_element_type=_F32) + l1b_ref[...]
    h = jnp.maximum(h, 0.0)
    h = jnp.dot(h.astype(_BF16), l2h_ref[...],
                preferred_element_type=_F32) + l2b_ref[...]
    h = jnp.maximum(h, 0.0)
    hb = h.astype(_BF16).astype(_F32)
    wb = l3r_ref[...].astype(_BF16).astype(_F32)
    return jnp.sum(hb * wb, axis=1, keepdims=True) + l3b_ref[...]


def _tailmeta(g):
    inA = (g >= 3) & (g <= 17) & (g % 2 == 1)
    inB = (g >= 18) & (g < 42)
    inC = (g >= 43) & (g <= 57) & (g % 2 == 1)
    inD = (g >= _D0 + 1) & (g < _G)
    lp = jnp.where(inB, 1 + jnp.clip(g - 18, 0, 23) // _MB,
                   jnp.where(inD, 1 + jnp.clip(g - _D0 - 1, 0, 23) // _MB, 0))
    mp = jnp.where(inA, jnp.clip((g - 3) // 2, 0, _MB - 1),
                   jnp.where(inB, jnp.clip(g - 18, 0, 23) % _MB,
                             jnp.where(inC, jnp.clip((g - 43) // 2, 0, _MB - 1),
                                       jnp.clip(g - _D0 - 1, 0, 23) % _MB)))
    bp = jnp.where(inC | inD, 1, 0)
    return lp, mp, bp, inA | inB | inC | inD


def _body(adj1_ref, adj2_ref, w1_ref, wn_ref, l1h_ref, l1b_ref,
          l2h_ref, l2b_ref, l3r_ref, l3b_ref,
          prod_ref, adj_s, x_s, rhs_s, s_s):
    g = pl.program_id(0)

    @pl.when(g < 16)
    def _():
        adj_s[pl.ds(g * _TF, _TF), :] = adj1_ref[...].astype(_BF16)

    lp, mp, bp, tv = _tailmeta(g)
    x = x_s[...]
    xn = _normalize(x)
    xs = jnp.where(lp <= 2, xn, x)
    sc = _score(xs, l1h_ref, l1b_ref, l2h_ref, l2b_ref, l3r_ref, l3b_ref)
    r = jnp.dot(xn.astype(_BF16), wn_ref[0].astype(_BF16),
                preferred_element_type=_F32)
    kslot = (bp * _MB + mp) * _TM

    @pl.when(tv & (lp == 0))
    def _():
        s_s[pl.ds(kslot, _TM), :] = sc

    @pl.when(tv & (lp >= 1) & (lp <= 2))
    def _():
        s_s[pl.ds(kslot, _TM), :] = s_s[pl.ds(kslot, _TM), :] + sc

    @pl.when(tv & (lp <= 2))
    def _():
        woff = ((lp + 1) % 2) * _N + mp * _TM
        rhs_s[pl.ds(woff, _TM), :] = r.astype(_BF16)

    @pl.when(tv & (lp == 3))
    def _():
        tot = s_s[pl.ds(kslot, _TM), :] + sc

        @pl.when(bp == 0)
        def _():
            s_s[pl.ds(kslot, _TM), :] = tot

        @pl.when(bp == 1)
        def _():
            prod_ref[...] = tot * s_s[pl.ds(mp * _TM, _TM), :]

    @pl.when((g >= 2) & (g <= 16) & (g % 2 == 0))
    def _():
        m = jnp.clip((g - 2) // 2, 0, _MB - 1)
        adjb = adj_s[pl.ds(m * _TM, _TM), :]
        x_s[...] = jnp.maximum(
            jnp.dot(adjb, w1_ref[...].astype(_BF16),
                    preferred_element_type=_F32), 0.0)

    @pl.when((g >= _C0) & (g < _C0 + 16))
    def _():
        j = jnp.clip(g - _C0, 0, 15)
        adjh = adj2_ref[...].astype(_BF16)
        adj_s[pl.ds(j * _TF, _TF), :] = adjh
        xh = jnp.maximum(
            jnp.dot(adjh, w1_ref[...].astype(_BF16),
                    preferred_element_type=_F32), 0.0)
        x_s[pl.ds((j % 2) * _TF, _TF), :] = xh

    inB = (g >= 17) & (g < _C0)
    inD = (g >= _D0) & (g < _D0 + 24)

    @pl.when(inB | inD)
    def _():
        q = jnp.where(inB, jnp.clip(g - 17, 0, 23), jnp.clip(g - _D0, 0, 23))
        l = 1 + q // _MB
        m = q % _MB
        adjb = adj_s[pl.ds(m * _TM, _TM), :]
        rhs = rhs_s[pl.ds((l % 2) * _N, _N), :]
        x_s[...] = jnp.maximum(
            jnp.dot(adjb, rhs, preferred_element_type=_F32), 0.0)


def _wn_idx(g):
    inB = (g >= 18) & (g < 42)
    inD = (g >= _D0 + 1)
    lw = jnp.where(inB, 1 + jnp.clip(g - 18, 0, 23) // _MB,
                   jnp.where(inD, 1 + jnp.clip(g - _D0 - 1, 0, 23) // _MB, 0))
    return jnp.clip(lw, 0, 2)


def kernel(adjacent_1, adjacent_2, W1, W2, W3, W4,
           lin1_w, lin1_b, lin2_w, lin2_b, lin3_w, lin3_b):
    a1_spec = pl.BlockSpec((_TF, _N), lambda g: (jnp.clip(g, 0, 15), 0))
    a2_spec = pl.BlockSpec((_TF, _N), lambda g: (jnp.clip(g - _C0, 0, 15), 0))
    wn_spec = pl.BlockSpec((1, _H, _H), lambda g: (_wn_idx(g), 0, 0))
    full = lambda shape: pl.BlockSpec(shape, lambda g: (0, 0))
    prod_spec = pl.BlockSpec(
        (_TM, 1), lambda g: (jnp.clip(g - (_D0 + 17), 0, _MB - 1), 0))
    wstack = jnp.stack([W2, W3, W4])
    lins = (
        lin1_w.astype(_BF16),
        lin1_b.reshape(1, 2 * _H),
        lin2_w.astype(_BF16),
        lin2_b.reshape(1, 2 * _H),
        lin3_w.reshape(1, 2 * _H),
        lin3_b.reshape(1, 1),
    )
    return pl.pallas_call(
        _body,
        grid=(_G,),
        in_specs=[a1_spec, a2_spec, full((_N, _H)), wn_spec,
                  full((_H, 2 * _H)), full((1, 2 * _H)),
                  full((2 * _H, 2 * _H)), full((1, 2 * _H)),
                  full((1, 2 * _H)), full((1, 1))],
        out_specs=prod_spec,
        out_shape=jax.ShapeDtypeStruct((_N, 1), _F32),
        compiler_params=pltpu.CompilerParams(
            vmem_limit_bytes=100 * 1024 * 1024),
        scratch_shapes=[
            pltpu.VMEM((_N, _N), _BF16),
            pltpu.VMEM((_TM, _H), _F32),
            pltpu.VMEM((2 * _N, _H), _BF16),
            pltpu.VMEM((2 * _N, 1), _F32),
        ],
    )(adjacent_1, adjacent_2, W1, wstack, *lins)


# speedup vs baseline: 1.3455x; 1.0962x over previous
import jax
import jax.numpy as jnp
from jax.experimental import pallas as pl
from jax.experimental.pallas import tpu as pltpu

_N = 4096
_H = 128
_TM = 1024
_MB = _N // _TM
_TF = 256
_B0 = 17
_C0 = 29
_D0 = 46
_G = _D0 + 12 + 1
_BF16 = jnp.bfloat16
_F32 = jnp.float32


def _normalize(x):
    n = jnp.sqrt(jnp.sum(x * x, axis=1, keepdims=True))
    return x / jnp.maximum(n, 1e-12)


def _score(x, l1h_ref, l1b_ref, l2h_ref, l2b_ref, l3r_ref, l3b_ref):
    h = jnp.dot(x.astype(_BF16), l1h_ref[...],
                preferred_element_type=_F32) + l1b_ref[...]
    h = jnp.maximum(h, 0.0)
    h = jnp.dot(h.astype(_BF16), l2h_ref[...],
                preferred_element_type=_F32) + l2b_ref[...]
    h = jnp.maximum(h, 0.0)
    hb = h.astype(_BF16).astype(_F32)
    wb = l3r_ref[...].astype(_BF16).astype(_F32)
    return jnp.sum(hb * wb, axis=1, keepdims=True) + l3b_ref[...]


def _tailmeta(g):
    inA = (g >= 5) & (g <= _B0) & (g % 4 == 1)
    inB = (g >= _B0 + 1) & (g < _C0 + 1)
    inC = (g >= _C0 + 4) & (g <= _D0 - 1) & ((g - _C0 - 4) % 4 == 0)
    inD = (g >= _D0 + 1) & (g < _G)
    lp = jnp.where(inB, 1 + jnp.clip(g - _B0 - 1, 0, 11) // _MB,
                   jnp.where(inD, 1 + jnp.clip(g - _D0 - 1, 0, 11) // _MB, 0))
    mp = jnp.where(inA, jnp.clip((g - 5) // 4, 0, _MB - 1),
                   jnp.where(inB, jnp.clip(g - _B0 - 1, 0, 11) % _MB,
                             jnp.where(inC,
                                       jnp.clip((g - _C0 - 4) // 4, 0, _MB - 1),
                                       jnp.clip(g - _D0 - 1, 0, 11) % _MB)))
    bp = jnp.where(inC | inD, 1, 0)
    return lp, mp, bp, inA | inB | inC | inD


def _body(adj1_ref, adj2_ref, w1_ref, wn_ref, l1h_ref, l1b_ref,
          l2h_ref, l2b_ref, l3r_ref, l3b_ref,
          prod_ref, adj_s, x_s, rhs_s, s_s):
    g = pl.program_id(0)

    @pl.when(g < 16)
    def _():
        adj_s[pl.ds(g * _TF, _TF), :] = adj1_ref[...].astype(_BF16)

    lp, mp, bp, tv = _tailmeta(g)
    x = x_s[...]
    xn = _normalize(x)
    xs = jnp.where(lp <= 2, xn, x)
    sc = _score(xs, l1h_ref, l1b_ref, l2h_ref, l2b_ref, l3r_ref, l3b_ref)
    r = jnp.dot(xn.astype(_BF16), wn_ref[0].astype(_BF16),
                preferred_element_type=_F32)
    kslot = (bp * _MB + mp) * _TM

    @pl.when(tv & (lp == 0))
    def _():
        s_s[pl.ds(kslot, _TM), :] = sc

    @pl.when(tv & (lp >= 1) & (lp <= 2))
    def _():
        s_s[pl.ds(kslot, _TM), :] = s_s[pl.ds(kslot, _TM), :] + sc

    @pl.when(tv & (lp <= 2))
    def _():
        woff = ((lp + 1) % 2) * _N + mp * _TM
        rhs_s[pl.ds(woff, _TM), :] = r.astype(_BF16)

    @pl.when(tv & (lp == 3))
    def _():
        tot = s_s[pl.ds(kslot, _TM), :] + sc

        @pl.when(bp == 0)
        def _():
            s_s[pl.ds(kslot, _TM), :] = tot

        @pl.when(bp == 1)
        def _():
            prod_ref[...] = tot * s_s[pl.ds(mp * _TM, _TM), :]

    @pl.when((g >= 4) & (g <= 16) & (g % 4 == 0))
    def _():
        m = jnp.clip(g // 4 - 1, 0, _MB - 1)
        adjb = adj_s[pl.ds(m * _TM, _TM), :]
        x_s[...] = jnp.maximum(
            jnp.dot(adjb, w1_ref[...].astype(_BF16),
                    preferred_element_type=_F32), 0.0)

    @pl.when((g >= _C0) & (g < _C0 + 16))
    def _():
        j = jnp.clip(g - _C0, 0, 15)
        adjh = adj2_ref[...].astype(_BF16)
        adj_s[pl.ds(j * _TF, _TF), :] = adjh
        xh = jnp.maximum(
            jnp.dot(adjh, w1_ref[...].astype(_BF16),
                    preferred_element_type=_F32), 0.0)
        x_s[pl.ds((j % 4) * _TF, _TF), :] = xh

    inBd = (g >= _B0) & (g < _B0 + 12)
    inDd = (g >= _D0) & (g < _D0 + 12)

    @pl.when(inBd | inDd)
    def _():
        q = jnp.where(inBd, jnp.clip(g - _B0, 0, 11), jnp.clip(g - _D0, 0, 11))
        l = 1 + q // _MB
        m = q % _MB
        adjb = adj_s[pl.ds(m * _TM, _TM), :]
        rhs = rhs_s[pl.ds((l % 2) * _N, _N), :]
        x_s[...] = jnp.maximum(
            jnp.dot(adjb, rhs, preferred_element_type=_F32), 0.0)


def _wn_idx(g):
    inB = (g >= _B0 + 1) & (g < _C0 + 1)
    inD = (g >= _D0 + 1)
    lw = jnp.where(inB, 1 + jnp.clip(g - _B0 - 1, 0, 11) // _MB,
                   jnp.where(inD, 1 + jnp.clip(g - _D0 - 1, 0, 11) // _MB, 0))
    return jnp.clip(lw, 0, 2)


def kernel(adjacent_1, adjacent_2, W1, W2, W3, W4,
           lin1_w, lin1_b, lin2_w, lin2_b, lin3_w, lin3_b):
    a1_spec = pl.BlockSpec((_TF, _N), lambda g: (jnp.clip(g, 0, 15), 0))
    a2_spec = pl.BlockSpec((_TF, _N), lambda g: (jnp.clip(g - _C0, 0, 15), 0))
    wn_spec = pl.BlockSpec((1, _H, _H), lambda g: (_wn_idx(g), 0, 0))
    full = lambda shape: pl.BlockSpec(shape, lambda g: (0, 0))
    prod_spec = pl.BlockSpec(
        (_TM, 1), lambda g: (jnp.clip(g - (_D0 + 9), 0, _MB - 1), 0))
    wstack = jnp.stack([W2, W3, W4])
    lins = (
        lin1_w.astype(_BF16),
        lin1_b.reshape(1, 2 * _H),
        lin2_w.astype(_BF16),
        lin2_b.reshape(1, 2 * _H),
        lin3_w.reshape(1, 2 * _H),
        lin3_b.reshape(1, 1),
    )
    return pl.pallas_call(
        _body,
        grid=(_G,),
        in_specs=[a1_spec, a2_spec, full((_N, _H)), wn_spec,
                  full((_H, 2 * _H)), full((1, 2 * _H)),
                  full((2 * _H, 2 * _H)), full((1, 2 * _H)),
                  full((1, 2 * _H)), full((1, 1))],
        out_specs=prod_spec,
        out_shape=jax.ShapeDtypeStruct((_N, 1), _F32),
        compiler_params=pltpu.CompilerParams(
            vmem_limit_bytes=100 * 1024 * 1024),
        scratch_shapes=[
            pltpu.VMEM((_N, _N), _BF16),
            pltpu.VMEM((_TM, _H), _F32),
            pltpu.VMEM((2 * _N, _H), _BF16),
            pltpu.VMEM((2 * _N, 1), _F32),
        ],
    )(adjacent_1, adjacent_2, W1, wstack, *lins)
